# baseline (speedup 1.0000x reference)
import functools

import jax
import jax.numpy as jnp
from jax import lax
from jax.experimental import pallas as pl
from jax.experimental.pallas import tpu as pltpu
from jax.experimental.pallas import tpu_sc as plsc

NT = 16
NC = 2
K = 128
KP = 64


def _fill_f32(ref, val, n16):
    def body(i, _):
        ref[pl.ds(i * 16, 16)] = jnp.full((16,), val, jnp.float32)
        return 0
    lax.fori_loop(0, n16, body, 0)


@functools.lru_cache(maxsize=None)
def _sc_deg(NP, E_PAD):
    per_w = E_PAD // (NC * NT)
    nch = per_w // K
    rows_t = NP // NT
    mesh = plsc.VectorSubcoreMesh(core_axis_name="c", subcore_axis_name="s")

    @functools.partial(
        pl.kernel,
        out_type=jax.ShapeDtypeStruct((NC, NP), jnp.float32),
        mesh=mesh,
        scratch_types=[
            pltpu.VMEM_SHARED((NP,), jnp.float32),
            pltpu.VMEM((nch, K), jnp.int32),
            pltpu.VMEM((K,), jnp.float32),
            pltpu.VMEM((rows_t,), jnp.float32),
        ],
    )
    def deg_kernel(dst_hbm, out_hbm, dacc, dst2, ones_v, zbuf):
        c = lax.axis_index("c")
        s = lax.axis_index("s")
        w = c * NT + s
        _fill_f32(zbuf, 0.0, rows_t // 16)
        _fill_f32(ones_v, 1.0, K // 16)
        pltpu.sync_copy(dst_hbm.at[w], dst2)
        pltpu.sync_copy(zbuf, dacc.at[pl.ds(s * rows_t, rows_t)])
        plsc.subcore_barrier()

        def chunk(j, _):
            pltpu.sync_copy(ones_v, dacc.at[dst2.at[j]], add=True)
            return 0
        lax.fori_loop(0, nch, chunk, 0)
        plsc.subcore_barrier()
        pltpu.sync_copy(dacc.at[pl.ds(s * rows_t, rows_t)],
                        out_hbm.at[c, pl.ds(s * rows_t, rows_t)])

    return deg_kernel


@functools.lru_cache(maxsize=None)
def _sc_prop(NP, E_PAD, F):
    per_t = E_PAD // NT
    nch = per_t // KP
    nbuf = 4
    nph = 4
    hch = nch // nph
    rows_t = NP // NT
    mesh = plsc.VectorSubcoreMesh(core_axis_name="c", subcore_axis_name="s")

    @functools.partial(
        pl.kernel,
        out_type=(jax.ShapeDtypeStruct((NP, F), jnp.float32),
                  jax.ShapeDtypeStruct((NP, F), jnp.float32)),
        mesh=mesh,
        scratch_types=[
            pltpu.VMEM_SHARED((NP, F), jnp.float32),
            pltpu.VMEM((hch, KP), jnp.int32),
            pltpu.VMEM((hch, KP), jnp.int32),
            pltpu.VMEM((nbuf, KP, F), jnp.float32),
            [pltpu.SemaphoreType.DMA] * nbuf,
        ],
    )
    def prop_kernel(h0, h1, src3, dst3, g0, g1, acc, src2, dst2, rows, sems):
        c = lax.axis_index("c")
        s = lax.axis_index("s")
        r0 = s * rows_t

        def run(tbl, out):
            pltpu.sync_copy(src3.at[s, pl.ds(0, hch)], src2)
            for b in range(nbuf):
                pltpu.async_copy(tbl.at[src2.at[b]], rows.at[b], sems[b])
            pltpu.sync_copy(dst3.at[s, pl.ds(0, hch)], dst2)
            pltpu.sync_copy(tbl.at[pl.ds(r0, rows_t)], acc.at[pl.ds(r0, rows_t)])
            plsc.subcore_barrier()
            for p in range(nph):
                if p > 0:
                    pltpu.sync_copy(src3.at[s, pl.ds(p * hch, hch)], src2)
                    pltpu.sync_copy(dst3.at[s, pl.ds(p * hch, hch)], dst2)
                    for b in range(nbuf):
                        pltpu.async_copy(tbl.at[src2.at[b]], rows.at[b], sems[b])

                def quad(jo, _):
                    for b in range(nbuf):
                        j = jo * nbuf + b
                        pltpu.make_async_copy(tbl.at[src2.at[j]],
                                              rows.at[b], sems[b]).wait()
                        pltpu.sync_copy(rows.at[b], acc.at[dst2.at[j]], add=True)

                        @pl.when(j + nbuf < hch)
                        def _():
                            pltpu.async_copy(tbl.at[src2.at[j + nbuf]],
                                             rows.at[b], sems[b])
                    return 0
                lax.fori_loop(0, hch // nbuf, quad, 0)
            plsc.subcore_barrier()
            pltpu.sync_copy(acc.at[pl.ds(r0, rows_t)], out.at[pl.ds(r0, rows_t)])

        @pl.when(c == 0)
        def _():
            run(h0, g0)

        @pl.when(c == 1)
        def _():
            run(h1, g1)

    return prop_kernel


def _dinv(deg_ref):
    return lax.rsqrt(deg_ref[:, 0:1] + deg_ref[:, 1:2] + 1.0)


def _tc_first_body(x_ref, w_ref, deg_ref, h0_ref, h1_ref):
    dinv = _dinv(deg_ref)
    h = lax.dot_general(x_ref[...].astype(jnp.bfloat16),
                        w_ref[...].astype(jnp.bfloat16),
                        (((1,), (1,)), ((), ())),
                        preferred_element_type=jnp.float32)
    hp = h * dinv
    half = hp.shape[1] // 2
    h0_ref[...] = hp[:, :half]
    h1_ref[...] = hp[:, half:]


def _tc_mid_body(g0_ref, g1_ref, deg_ref, b_ref, w_ref, h0_ref, h1_ref):
    dinv = _dinv(deg_ref)
    g = jnp.concatenate([g0_ref[...], g1_ref[...]],
                        axis=1).astype(jnp.float32)
    u = jnp.maximum(g * dinv + b_ref[...], 0.0)
    h = lax.dot_general(u.astype(jnp.bfloat16),
                        w_ref[...].astype(jnp.bfloat16),
                        (((1,), (1,)), ((), ())),
                        preferred_element_type=jnp.float32)
    hp = h * dinv
    half = hp.shape[1] // 2
    h0_ref[...] = hp[:, :half]
    h1_ref[...] = hp[:, half:]


def _tc_final_body(g0_ref, g1_ref, deg_ref, b_ref, w_ref, bl_ref, o_ref):
    dinv = _dinv(deg_ref)
    g = jnp.concatenate([g0_ref[...], g1_ref[...]],
                        axis=1).astype(jnp.float32)
    u = jnp.maximum(g * dinv + b_ref[...], 0.0)
    o_ref[...] = lax.dot_general(u.astype(jnp.bfloat16),
                                 w_ref[...].astype(jnp.bfloat16),
                                 (((1,), (1,)), ((), ())),
                                 preferred_element_type=jnp.float32) + bl_ref[...]


@functools.lru_cache(maxsize=None)
def _tc_first(NP, D, BN, n):
    return pl.pallas_call(
        _tc_first_body,
        grid=(n // BN,),
        in_specs=[
            pl.BlockSpec((BN, D), lambda i: (i, 0)),
            pl.BlockSpec((D, D), lambda i: (0, 0)),
            pl.BlockSpec((BN, 2), lambda i: (i, 0)),
        ],
        out_specs=(pl.BlockSpec((BN, D // 2), lambda i: (i, 0)),
                   pl.BlockSpec((BN, D // 2), lambda i: (i, 0))),
        out_shape=(jax.ShapeDtypeStruct((NP, D // 2), jnp.float32),
                   jax.ShapeDtypeStruct((NP, D // 2), jnp.float32)),
    )


@functools.lru_cache(maxsize=None)
def _tc_mid(NP, D, BN, n):
    return pl.pallas_call(
        _tc_mid_body,
        grid=(n // BN,),
        in_specs=[
            pl.BlockSpec((BN, D // 2), lambda i: (i, 0)),
            pl.BlockSpec((BN, D // 2), lambda i: (i, 0)),
            pl.BlockSpec((BN, 2), lambda i: (i, 0)),
            pl.BlockSpec((1, D), lambda i: (0, 0)),
            pl.BlockSpec((D, D), lambda i: (0, 0)),
        ],
        out_specs=(pl.BlockSpec((BN, D // 2), lambda i: (i, 0)),
                   pl.BlockSpec((BN, D // 2), lambda i: (i, 0))),
        out_shape=(jax.ShapeDtypeStruct((NP, D // 2), jnp.float32),
                   jax.ShapeDtypeStruct((NP, D // 2), jnp.float32)),
    )


@functools.lru_cache(maxsize=None)
def _tc_final(NP, D, BN, n):
    return pl.pallas_call(
        _tc_final_body,
        grid=(n // BN,),
        in_specs=[
            pl.BlockSpec((BN, D // 2), lambda i: (i, 0)),
            pl.BlockSpec((BN, D // 2), lambda i: (i, 0)),
            pl.BlockSpec((BN, 2), lambda i: (i, 0)),
            pl.BlockSpec((1, D), lambda i: (0, 0)),
            pl.BlockSpec((D, D), lambda i: (0, 0)),
            pl.BlockSpec((1, D), lambda i: (0, 0)),
        ],
        out_specs=pl.BlockSpec((BN, D), lambda i: (i, 0)),
        out_shape=jax.ShapeDtypeStruct((n, D), jnp.float32),
    )


def kernel(x, edge_index, W1, b1, W2, b2, W3, b3, Wl, bl):
    n, d = x.shape
    e = edge_index.shape[1]
    NP = ((n + 2047) // 2048) * 2048
    if NP == n:
        NP += 2048
    gran = NT * KP * 4 * 2
    E_PAD = ((e + gran - 1) // gran) * gran
    pad_e = E_PAD - e

    src = edge_index[0]
    dst = edge_index[1]
    fill = n + (jnp.arange(pad_e, dtype=jnp.int32) % (NP - n))
    src_p = jnp.concatenate([src, fill])
    dst_p = jnp.concatenate([dst, fill])

    deg2 = _sc_deg(NP, E_PAD)(dst_p.reshape(NC * NT, -1, K))
    degT = deg2.T

    BN = 1000
    first = _tc_first(NP, d, BN, n)
    mid = _tc_mid(NP, d, BN, n)
    final = _tc_final(NP, d, BN, n)
    prop = _sc_prop(NP, E_PAD, d // 2)
    src3 = src_p.reshape(NT, -1, KP)
    dst3 = dst_p.reshape(NT, -1, KP)

    h0, h1 = first(x, W1, degT)
    g0, g1 = prop(h0, h1, src3, dst3)
    h0, h1 = mid(g0, g1, degT, b1.reshape(1, -1), W2)
    g0, g1 = prop(h0, h1, src3, dst3)
    h0, h1 = mid(g0, g1, degT, b2.reshape(1, -1), W3)
    g0, g1 = prop(h0, h1, src3, dst3)
    return final(g0, g1, degT, b3.reshape(1, -1), Wl, bl.reshape(1, -1))


# speedup vs baseline: 1.0485x; 1.0485x over previous
import functools

import jax
import jax.numpy as jnp
from jax import lax
from jax.experimental import pallas as pl
from jax.experimental.pallas import tpu as pltpu
from jax.experimental.pallas import tpu_sc as plsc

NT = 16
NC = 2
K = 128
KP = 64


def _fill_f32(ref, val, n16):
    def body(i, _):
        ref[pl.ds(i * 16, 16)] = jnp.full((16,), val, jnp.float32)
        return 0
    lax.fori_loop(0, n16, body, 0)


@functools.lru_cache(maxsize=None)
def _sc_deg(NP, E_PAD):
    per_w = E_PAD // (NC * NT)
    nch = per_w // K
    rows_t = NP // NT
    mesh = plsc.VectorSubcoreMesh(core_axis_name="c", subcore_axis_name="s")

    @functools.partial(
        pl.kernel,
        out_type=jax.ShapeDtypeStruct((NC, NP), jnp.float32),
        mesh=mesh,
        scratch_types=[
            pltpu.VMEM_SHARED((NP,), jnp.float32),
            pltpu.VMEM((nch, K), jnp.int32),
            pltpu.VMEM((K,), jnp.float32),
            pltpu.VMEM((rows_t,), jnp.float32),
        ],
    )
    def deg_kernel(dst_hbm, out_hbm, dacc, dst2, ones_v, zbuf):
        c = lax.axis_index("c")
        s = lax.axis_index("s")
        w = c * NT + s
        _fill_f32(zbuf, 0.0, rows_t // 16)
        _fill_f32(ones_v, 1.0, K // 16)
        pltpu.sync_copy(dst_hbm.at[w], dst2)
        pltpu.sync_copy(zbuf, dacc.at[pl.ds(s * rows_t, rows_t)])
        plsc.subcore_barrier()

        def chunk(j, _):
            pltpu.sync_copy(ones_v, dacc.at[dst2.at[j]], add=True)
            return 0
        lax.fori_loop(0, nch, chunk, 0)
        plsc.subcore_barrier()
        pltpu.sync_copy(dacc.at[pl.ds(s * rows_t, rows_t)],
                        out_hbm.at[c, pl.ds(s * rows_t, rows_t)])

    return deg_kernel


@functools.lru_cache(maxsize=None)
def _sc_prop(NP, E_PAD, F):
    per_t = E_PAD // NT
    nch = per_t // KP
    nbuf = 4
    nph = 4
    hch = nch // nph
    rows_t = NP // NT
    mesh = plsc.VectorSubcoreMesh(core_axis_name="c", subcore_axis_name="s")

    @functools.partial(
        pl.kernel,
        out_type=(jax.ShapeDtypeStruct((NP, F), jnp.float32),
                  jax.ShapeDtypeStruct((NP, F), jnp.float32)),
        mesh=mesh,
        scratch_types=[
            pltpu.VMEM_SHARED((NP, F), jnp.float32),
            pltpu.VMEM((hch, KP), jnp.int32),
            pltpu.VMEM((hch, KP), jnp.int32),
            pltpu.VMEM((nbuf, KP, F), jnp.float32),
            [pltpu.SemaphoreType.DMA] * nbuf,
        ],
    )
    def prop_kernel(h0, h1, src3, dst3, g0, g1, acc, src2, dst2, rows, sems):
        c = lax.axis_index("c")
        s = lax.axis_index("s")
        r0 = s * rows_t

        def run(tbl, out):
            pltpu.sync_copy(src3.at[s, pl.ds(0, hch)], src2)
            for b in range(nbuf):
                pltpu.async_copy(tbl.at[src2.at[b]], rows.at[b], sems[b])
            pltpu.sync_copy(dst3.at[s, pl.ds(0, hch)], dst2)
            pltpu.sync_copy(tbl.at[pl.ds(r0, rows_t)], acc.at[pl.ds(r0, rows_t)])
            plsc.subcore_barrier()
            for p in range(nph):
                if p > 0:
                    pltpu.sync_copy(src3.at[s, pl.ds(p * hch, hch)], src2)
                    pltpu.sync_copy(dst3.at[s, pl.ds(p * hch, hch)], dst2)
                    for b in range(nbuf):
                        pltpu.async_copy(tbl.at[src2.at[b]], rows.at[b], sems[b])

                def quad(jo, _):
                    for b in range(nbuf):
                        j = jo * nbuf + b
                        pltpu.make_async_copy(tbl.at[src2.at[j]],
                                              rows.at[b], sems[b]).wait()
                        pltpu.sync_copy(rows.at[b], acc.at[dst2.at[j]], add=True)

                        @pl.when(j + nbuf < hch)
                        def _():
                            pltpu.async_copy(tbl.at[src2.at[j + nbuf]],
                                             rows.at[b], sems[b])
                    return 0
                lax.fori_loop(0, hch // nbuf, quad, 0)
            plsc.subcore_barrier()
            pltpu.sync_copy(acc.at[pl.ds(r0, rows_t)], out.at[pl.ds(r0, rows_t)])

        @pl.when(c == 0)
        def _():
            run(h0, g0)

        @pl.when(c == 1)
        def _():
            run(h1, g1)

    return prop_kernel


def _dinv(deg_ref):
    return lax.rsqrt(deg_ref[:, 0:1] + deg_ref[:, 1:2] + 1.0)


def _tc_first_body(x_ref, w_ref, deg_ref, h0_ref, h1_ref):
    dinv = _dinv(deg_ref)
    h = lax.dot_general(x_ref[...].astype(jnp.bfloat16),
                        w_ref[...].astype(jnp.bfloat16),
                        (((1,), (1,)), ((), ())),
                        preferred_element_type=jnp.float32)
    hp = h * dinv
    half = hp.shape[1] // 2
    h0_ref[...] = hp[:, :half]
    h1_ref[...] = hp[:, half:]


def _tc_mid_body(g0_ref, g1_ref, deg_ref, b_ref, w_ref, h0_ref, h1_ref):
    dinv = _dinv(deg_ref)
    g = jnp.concatenate([g0_ref[...], g1_ref[...]],
                        axis=1).astype(jnp.float32)
    u = jnp.maximum(g * dinv + b_ref[...], 0.0)
    h = lax.dot_general(u.astype(jnp.bfloat16),
                        w_ref[...].astype(jnp.bfloat16),
                        (((1,), (1,)), ((), ())),
                        preferred_element_type=jnp.float32)
    hp = h * dinv
    half = hp.shape[1] // 2
    h0_ref[...] = hp[:, :half]
    h1_ref[...] = hp[:, half:]


def _tc_final_body(g0_ref, g1_ref, deg_ref, b_ref, w_ref, bl_ref, o_ref):
    dinv = _dinv(deg_ref)
    g = jnp.concatenate([g0_ref[...], g1_ref[...]],
                        axis=1).astype(jnp.float32)
    u = jnp.maximum(g * dinv + b_ref[...], 0.0)
    o_ref[...] = lax.dot_general(u.astype(jnp.bfloat16),
                                 w_ref[...].astype(jnp.bfloat16),
                                 (((1,), (1,)), ((), ())),
                                 preferred_element_type=jnp.float32) + bl_ref[...]


@functools.lru_cache(maxsize=None)
def _tc_first(NP, D, BN, n):
    return pl.pallas_call(
        _tc_first_body,
        grid=(n // BN,),
        in_specs=[
            pl.BlockSpec((BN, D), lambda i: (i, 0)),
            pl.BlockSpec((D, D), lambda i: (0, 0)),
            pl.BlockSpec((BN, 2), lambda i: (i, 0)),
        ],
        out_specs=(pl.BlockSpec((BN, D // 2), lambda i: (i, 0)),
                   pl.BlockSpec((BN, D // 2), lambda i: (i, 0))),
        out_shape=(jax.ShapeDtypeStruct((NP, D // 2), jnp.float32),
                   jax.ShapeDtypeStruct((NP, D // 2), jnp.float32)),
    )


@functools.lru_cache(maxsize=None)
def _tc_mid(NP, D, BN, n):
    return pl.pallas_call(
        _tc_mid_body,
        grid=(n // BN,),
        in_specs=[
            pl.BlockSpec((BN, D // 2), lambda i: (i, 0)),
            pl.BlockSpec((BN, D // 2), lambda i: (i, 0)),
            pl.BlockSpec((BN, 2), lambda i: (i, 0)),
            pl.BlockSpec((1, D), lambda i: (0, 0)),
            pl.BlockSpec((D, D), lambda i: (0, 0)),
        ],
        out_specs=(pl.BlockSpec((BN, D // 2), lambda i: (i, 0)),
                   pl.BlockSpec((BN, D // 2), lambda i: (i, 0))),
        out_shape=(jax.ShapeDtypeStruct((NP, D // 2), jnp.float32),
                   jax.ShapeDtypeStruct((NP, D // 2), jnp.float32)),
    )


@functools.lru_cache(maxsize=None)
def _tc_final(NP, D, BN, n):
    return pl.pallas_call(
        _tc_final_body,
        grid=(n // BN,),
        in_specs=[
            pl.BlockSpec((BN, D // 2), lambda i: (i, 0)),
            pl.BlockSpec((BN, D // 2), lambda i: (i, 0)),
            pl.BlockSpec((BN, 2), lambda i: (i, 0)),
            pl.BlockSpec((1, D), lambda i: (0, 0)),
            pl.BlockSpec((D, D), lambda i: (0, 0)),
            pl.BlockSpec((1, D), lambda i: (0, 0)),
        ],
        out_specs=pl.BlockSpec((BN, D), lambda i: (i, 0)),
        out_shape=jax.ShapeDtypeStruct((n, D), jnp.float32),
    )


def kernel(x, edge_index, W1, b1, W2, b2, W3, b3, Wl, bl):
    n, d = x.shape
    e = edge_index.shape[1]
    NP = ((n + 2047) // 2048) * 2048
    if NP == n:
        NP += 2048
    gran = NT * KP * 4 * 2
    E_PAD = ((e + gran - 1) // gran) * gran
    pad_e = E_PAD - e

    src = edge_index[0]
    dst = edge_index[1]
    fill = n + (jnp.arange(pad_e, dtype=jnp.int32) % (NP - n))
    src_p = jnp.concatenate([src, fill])
    dst_p = jnp.concatenate([dst, fill])

    deg2 = _sc_deg(NP, E_PAD)(dst_p.reshape(NC * NT, -1, K))
    degT = deg2.T

    BN = 5000
    first = _tc_first(NP, d, BN, n)
    mid = _tc_mid(NP, d, BN, n)
    final = _tc_final(NP, d, BN, n)
    prop = _sc_prop(NP, E_PAD, d // 2)
    src3 = src_p.reshape(NT, -1, KP)
    dst3 = dst_p.reshape(NT, -1, KP)

    h0, h1 = first(x, W1, degT)
    g0, g1 = prop(h0, h1, src3, dst3)
    h0, h1 = mid(g0, g1, degT, b1.reshape(1, -1), W2)
    g0, g1 = prop(h0, h1, src3, dst3)
    h0, h1 = mid(g0, g1, degT, b2.reshape(1, -1), W3)
    g0, g1 = prop(h0, h1, src3, dst3)
    return final(g0, g1, degT, b3.reshape(1, -1), Wl, bl.reshape(1, -1))
